# baseline (speedup 1.0000x reference)
import functools

import jax
import jax.numpy as jnp
import numpy as np
from jax import lax
from jax.experimental import pallas as pl
from jax.experimental.pallas import tpu as pltpu
from jax.experimental.pallas import tpu_sc as plsc

N = 50000
E = 800000
HID = 64
N_LAYERS = 4
NSEG = 500
SEGP = 512
PK = 128
EPAD = 802816
BE = 2048
BN = 1000

_CENTERS = np.linspace(-2.0, 2.0, 16, dtype=np.float32)
_GWIDTH = 4.0 / 16.0


def _silu(x):
    return x * jax.nn.sigmoid(x)


def _embed_body(mu_ref, t_ref, c_ref, w_ref, b_ref, out_ref):
    x = mu_ref[...]
    centers = (lax.broadcasted_iota(jnp.int32, (1, 16), 1).astype(jnp.float32)
               * (4.0 / 15.0) - 2.0)
    z = (x - centers) / _GWIDTH
    g = jnp.exp(-0.5 * z * z)
    g = g / jnp.clip(jnp.sum(jnp.abs(g), axis=1, keepdims=True), 1e-12, None)
    g = g * 2.0 - 1.0
    feat = jnp.concatenate([g, t_ref[...]], axis=1)
    h0 = jnp.dot(feat, w_ref[...], preferred_element_type=jnp.float32) + b_ref[...]
    out_ref[...] = jnp.concatenate(
        [h0, c_ref[...], jnp.zeros((BN, PK - 67), jnp.float32)], axis=1)


def _embed(mu_charge_t, time, mu_pos_t, w, b):
    grid = (N // BN,)
    return pl.pallas_call(
        _embed_body,
        grid=grid,
        in_specs=[
            pl.BlockSpec((BN, 1), lambda i: (i, 0)),
            pl.BlockSpec((BN, 1), lambda i: (i, 0)),
            pl.BlockSpec((BN, 3), lambda i: (i, 0)),
            pl.BlockSpec((17, HID), lambda i: (0, 0)),
            pl.BlockSpec((1, HID), lambda i: (0, 0)),
        ],
        out_specs=pl.BlockSpec((BN, PK), lambda i: (i, 0)),
        out_shape=jax.ShapeDtypeStruct((N, PK), jnp.float32),
    )(mu_charge_t, time, mu_pos_t, w, b.reshape(1, HID))


def _edge_body(g1_ref, g2_ref, w1_ref, w2_ref, wc1_ref, aux_ref,
               mlo_ref, mhi_ref, cp_ref):
    b = pl.program_id(0)
    g1 = g1_ref[...]
    g2 = g2_ref[...]
    diff = g1[:, 64:67] - g2[:, 64:67]
    radial = jnp.sum(diff * diff, axis=1, keepdims=True)
    cdiff = diff / (jnp.sqrt(radial) + 1.0)
    f = jnp.concatenate([g1, g2], axis=1)
    w1r = aux_ref[0:1, :]
    b1 = aux_ref[1:2, :]
    b2 = aux_ref[2:3, :]
    bc1 = aux_ref[3:4, :]
    wc2 = aux_ref[4:5, :]
    a1 = _silu(jnp.dot(f, w1_ref[...], preferred_element_type=jnp.float32)
               + radial * w1r + b1)
    m = _silu(jnp.dot(a1, w2_ref[...], preferred_element_type=jnp.float32) + b2)
    t = _silu(jnp.dot(m, wc1_ref[...], preferred_element_type=jnp.float32) + bc1)
    phi = jnp.sum(t * wc2, axis=1, keepdims=True)
    gidx = b * BE + lax.broadcasted_iota(jnp.int32, (BE, 1), 0)
    msk = (gidx < E).astype(jnp.float32)
    m = m * msk
    mlo_ref[...] = m[:, 0:32]
    mhi_ref[...] = m[:, 32:64]
    cp_ref[...] = jnp.concatenate(
        [cdiff * phi * msk, msk, jnp.zeros((BE, 12), jnp.float32)], axis=1)


def _edge_mlp(G, w1cat, w2, wc1, aux):
    nb = EPAD // BE
    return pl.pallas_call(
        _edge_body,
        grid=(nb,),
        in_specs=[
            pl.BlockSpec((BE, PK), lambda b: (b, 0)),
            pl.BlockSpec((BE, PK), lambda b, nb=nb: (b + nb, 0)),
            pl.BlockSpec((2 * PK, HID), lambda b: (0, 0)),
            pl.BlockSpec((HID, HID), lambda b: (0, 0)),
            pl.BlockSpec((HID, HID), lambda b: (0, 0)),
            pl.BlockSpec((8, HID), lambda b: (0, 0)),
        ],
        out_specs=[
            pl.BlockSpec((BE, 32), lambda b: (b, 0)),
            pl.BlockSpec((BE, 32), lambda b: (b, 0)),
            pl.BlockSpec((BE, 16), lambda b: (b, 0)),
        ],
        out_shape=[
            jax.ShapeDtypeStruct((EPAD, 32), jnp.float32),
            jax.ShapeDtypeStruct((EPAD, 32), jnp.float32),
            jax.ShapeDtypeStruct((EPAD, 16), jnp.float32),
        ],
    )(G, G, w1cat, w2, wc1, aux)


def _node_body(pk_ref, mlo_ref, mhi_ref, cg_ref, w1_ref, w2_ref, aux_ref, out_ref):
    pk = pk_ref[...]
    h = pk[:, 0:64]
    coord = pk[:, 64:67]
    cagg = cg_ref[...]
    cnt = jnp.clip(cagg[:, 3:4], 1.0, None)
    coord2 = coord + cagg[:, 0:3] / cnt
    f = jnp.concatenate([h, mlo_ref[...], mhi_ref[...]], axis=1)
    b1 = aux_ref[0:1, :]
    b2 = aux_ref[1:2, :]
    u = _silu(jnp.dot(f, w1_ref[...], preferred_element_type=jnp.float32) + b1)
    h2 = h + jnp.dot(u, w2_ref[...], preferred_element_type=jnp.float32) + b2
    out_ref[...] = jnp.concatenate(
        [h2, coord2, jnp.zeros((BN, PK - 67), jnp.float32)], axis=1)


def _node_mlp(packed, magg_lo, magg_hi, cagg, w1, w2, aux):
    return pl.pallas_call(
        _node_body,
        grid=(N // BN,),
        in_specs=[
            pl.BlockSpec((BN, PK), lambda i: (i, 0)),
            pl.BlockSpec((BN, 32), lambda i: (i, 0)),
            pl.BlockSpec((BN, 32), lambda i: (i, 0)),
            pl.BlockSpec((BN, 16), lambda i: (i, 0)),
            pl.BlockSpec((2 * HID, HID), lambda i: (0, 0)),
            pl.BlockSpec((HID, HID), lambda i: (0, 0)),
            pl.BlockSpec((8, HID), lambda i: (0, 0)),
        ],
        out_specs=pl.BlockSpec((BN, PK), lambda i: (i, 0)),
        out_shape=jax.ShapeDtypeStruct((N, PK), jnp.float32),
    )(packed, magg_lo, magg_hi, cagg, w1, w2, aux)


def _segA_body(ids_ref, pk_ref, mp_ref, acc_ref):
    b = pl.program_id(0)
    ids = ids_ref[0]
    eps0 = pk_ref[...][:, 64:67] - mp_ref[...]
    vals = jnp.concatenate(
        [eps0, jnp.ones((BN, 1), jnp.float32), jnp.zeros((BN, 4), jnp.float32)],
        axis=1)
    iota_col = lax.broadcasted_iota(jnp.int32, (SEGP, 1), 0)
    oh = (iota_col == ids).astype(jnp.float32)
    part = jnp.dot(oh, vals, preferred_element_type=jnp.float32)

    @pl.when(b == 0)
    def _():
        acc_ref[...] = part

    @pl.when(b > 0)
    def _():
        acc_ref[...] += part


def _seg_sums(ids3, packed, mu_pos_t):
    return pl.pallas_call(
        _segA_body,
        grid=(N // BN,),
        in_specs=[
            pl.BlockSpec((1, 1, BN), lambda i: (i, 0, 0)),
            pl.BlockSpec((BN, PK), lambda i: (i, 0)),
            pl.BlockSpec((BN, 3), lambda i: (i, 0)),
        ],
        out_specs=pl.BlockSpec((SEGP, 8), lambda i: (0, 0)),
        out_shape=jax.ShapeDtypeStruct((SEGP, 8), jnp.float32),
    )(ids3, packed, mu_pos_t)


def _segB_body(idsc_ref, pk_ref, mp_ref, mc_ref, gc_ref, gch_ref, ss_ref,
               aux_ref, cp_ref, kh_ref):
    ss = ss_ref[...]
    mean = ss[:, 0:3] / jnp.clip(ss[:, 3:4], 1.0, None)
    meanp = jnp.concatenate([mean, jnp.zeros((SEGP, 5), jnp.float32)], axis=1)
    iota_row = lax.broadcasted_iota(jnp.int32, (1, SEGP), 1)
    oh = (idsc_ref[...] == iota_row).astype(jnp.float32)
    mnode = jnp.dot(oh, meanp, preferred_element_type=jnp.float32)[:, 0:3]
    pk = pk_ref[...]
    mp = mp_ref[...]
    eps = jnp.clip(pk[:, 64:67] - mp - mnode, -10.0, 10.0)
    g = gc_ref[...]
    cp_ref[...] = mp / g - jnp.sqrt((1.0 - g) / g) * eps
    w0 = aux_ref[0:1, :]
    b0 = aux_ref[1:2, 0:1]
    mu_eps = jnp.clip(jnp.sum(pk[:, 0:64] * w0, axis=1, keepdims=True) + b0,
                      -10.0, 10.0)
    gch = gch_ref[...]
    kh_ref[...] = mc_ref[...] / gch - jnp.sqrt((1.0 - gch) / gch) * mu_eps


def _head(idsc, packed, mu_pos_t, mu_charge_t, gamma_coord, gamma_charge,
          seg_sums, aux):
    return pl.pallas_call(
        _segB_body,
        grid=(N // BN,),
        in_specs=[
            pl.BlockSpec((BN, 1), lambda i: (i, 0)),
            pl.BlockSpec((BN, PK), lambda i: (i, 0)),
            pl.BlockSpec((BN, 3), lambda i: (i, 0)),
            pl.BlockSpec((BN, 1), lambda i: (i, 0)),
            pl.BlockSpec((BN, 1), lambda i: (i, 0)),
            pl.BlockSpec((BN, 1), lambda i: (i, 0)),
            pl.BlockSpec((SEGP, 8), lambda i: (0, 0)),
            pl.BlockSpec((8, HID), lambda i: (0, 0)),
        ],
        out_specs=[
            pl.BlockSpec((BN, 3), lambda i: (i, 0)),
            pl.BlockSpec((BN, 1), lambda i: (i, 0)),
        ],
        out_shape=[
            jax.ShapeDtypeStruct((N, 3), jnp.float32),
            jax.ShapeDtypeStruct((N, 1), jnp.float32),
        ],
    )(idsc, packed, mu_pos_t, mu_charge_t, gamma_coord, gamma_charge,
      seg_sums, aux)


_GW = 128


def _gather(packed, idx2):
    n_idx = 2 * EPAD

    @functools.partial(
        pl.kernel,
        out_type=jax.ShapeDtypeStruct((n_idx, PK), jnp.float32),
        mesh=plsc.VectorSubcoreMesh(core_axis_name="c", subcore_axis_name="s"),
    )
    def k(x_hbm, i_hbm, o_hbm):
        def body(i_vmem, o_vmem):
            pltpu.sync_copy(x_hbm.at[i_vmem.at[0]], o_vmem)

        pltpu.emit_pipeline(
            body,
            grid=(n_idx // _GW,),
            in_specs=[pl.BlockSpec((1, _GW), index_map=lambda i: (0, i))],
            out_specs=[pl.BlockSpec((_GW, PK), index_map=lambda i: (i, 0))],
            core_axis_name=("c", "s"),
            dimension_semantics=(pltpu.PARALLEL,),
        )(i_hbm, o_hbm)

    return k(packed, idx2)


_NH = N // 2
_NA = 50048
_NB = 25088
_S1CH = 256
_S1SUP = EPAD // _S1CH
_S2CH = 512
_S2SUP = EPAD // _S2CH

_SC_PARAMS = pltpu.CompilerParams(use_tc_tiling_on_sc=False)


def _scatter_m(m_lo, m_hi, rows4, za):
    nper = _S1SUP // 16

    @functools.partial(
        pl.kernel,
        out_type=[
            jax.ShapeDtypeStruct((_NA, 32), jnp.float32),
            jax.ShapeDtypeStruct((_NA, 32), jnp.float32),
        ],
        mesh=plsc.VectorSubcoreMesh(core_axis_name="c", subcore_axis_name="s"),
        scratch_types=[
            pltpu.VMEM_SHARED((_NA, 32), jnp.float32),
            pltpu.VMEM((_S1CH // 128, 128), jnp.int32),
            pltpu.VMEM((_S1CH // 128, 128), jnp.int32),
            pltpu.VMEM((_S1CH, 32), jnp.float32),
            pltpu.VMEM((_S1CH, 32), jnp.float32),
            pltpu.SemaphoreType.DMA,
        ],
        compiler_params=_SC_PARAMS,
    )
    def k(mlo_hbm, mhi_hbm, rows_hbm, za_hbm, olo_hbm, ohi_hbm,
          accA, idx0, idx1, mb0, mb1, sem):
        c = lax.axis_index("c")
        s = lax.axis_index("s")
        pltpu.sync_copy(za_hbm, accA.at[pl.ds(s * (_NA // 16), _NA // 16)])
        plsc.subcore_barrier()

        def load(u, ib, mb):
            pltpu.async_copy(rows_hbm.at[u], ib, sem)

            @pl.when(c == 0)
            def _():
                pltpu.async_copy(mlo_hbm.at[pl.ds(u * _S1CH, _S1CH)], mb, sem)

            @pl.when(c == 1)
            def _():
                pltpu.async_copy(mhi_hbm.at[pl.ds(u * _S1CH, _S1CH)], mb, sem)

        def wait_load(u, ib, mb):
            pltpu.make_async_copy(rows_hbm.at[u], ib, sem).wait()
            pltpu.make_async_copy(mlo_hbm.at[pl.ds(u * _S1CH, _S1CH)], mb,
                                  sem).wait()

        def scat(ib, mb):
            @pl.loop(0, _S1CH // 128)
            def _(kk):
                pltpu.sync_copy(mb.at[pl.ds(kk * 128, 128)],
                                accA.at[ib.at[kk]], add=True)

        base = s * nper
        load(base, idx0, mb0)

        @pl.loop(0, nper, step=2)
        def _(j):
            u0 = base + j
            u1 = u0 + 1
            load(u1, idx1, mb1)
            wait_load(u0, idx0, mb0)
            scat(idx0, mb0)

            @pl.when(j + 2 < nper)
            def _():
                load(u0 + 2, idx0, mb0)

            wait_load(u1, idx1, mb1)
            scat(idx1, mb1)

        plsc.subcore_barrier()

        @pl.when(c == 0)
        def _():
            pltpu.sync_copy(accA.at[pl.ds(s * (_NA // 16), _NA // 16)],
                            olo_hbm.at[pl.ds(s * (_NA // 16), _NA // 16)])

        @pl.when(c == 1)
        def _():
            pltpu.sync_copy(accA.at[pl.ds(s * (_NA // 16), _NA // 16)],
                            ohi_hbm.at[pl.ds(s * (_NA // 16), _NA // 16)])

    return k(m_lo, m_hi, rows4, za)


def _scatter_c(cpack, rows8, zb):
    nper = _S2SUP // 16

    @functools.partial(
        pl.kernel,
        out_type=jax.ShapeDtypeStruct((2, _NB, 16), jnp.float32),
        mesh=plsc.VectorSubcoreMesh(core_axis_name="c", subcore_axis_name="s"),
        scratch_types=[
            pltpu.VMEM_SHARED((_NB, 16), jnp.float32),
            pltpu.VMEM((_S2CH // 128, 128), jnp.int32),
            pltpu.VMEM((_S2CH // 128, 128), jnp.int32),
            pltpu.VMEM((_S2CH // 128, 128), jnp.int32),
            pltpu.VMEM((_S2CH, 16), jnp.float32),
            pltpu.VMEM((_S2CH, 16), jnp.float32),
            pltpu.SemaphoreType.DMA,
        ],
        compiler_params=_SC_PARAMS,
    )
    def k(cp_hbm, rows_hbm, zb_hbm, ocg_hbm,
          accB, idx0, idx1, idxt, cb0, cb1, sem):
        c = lax.axis_index("c")
        s = lax.axis_index("s")
        pltpu.sync_copy(zb_hbm, accB.at[pl.ds(s * (_NB // 16), _NB // 16)])
        plsc.subcore_barrier()
        nbase = c * _NH

        def load(u, ib, cb):
            pltpu.async_copy(rows_hbm.at[u], ib, sem)
            pltpu.async_copy(cp_hbm.at[pl.ds(u * _S2CH, _S2CH)], cb, sem)

        def wait_load(u, ib, cb):
            pltpu.make_async_copy(rows_hbm.at[u], ib, sem).wait()
            pltpu.make_async_copy(cp_hbm.at[pl.ds(u * _S2CH, _S2CH)], cb,
                                  sem).wait()

        def scat(ib, cb):
            @pl.loop(0, _S2CH // 128)
            def _(kk):
                @pl.loop(0, 8)
                def _(t):
                    v = ib[kk, pl.ds(t * 16, 16)]
                    l = v - nbase
                    ok = (l >= 0) & (l < _NH)
                    idxt[kk, pl.ds(t * 16, 16)] = jnp.where(ok, l, _NB - 1)

            @pl.loop(0, _S2CH // 128)
            def _(kk):
                pltpu.sync_copy(cb.at[pl.ds(kk * 128, 128)],
                                accB.at[idxt.at[kk]], add=True)

        base = s * nper
        load(base, idx0, cb0)

        @pl.loop(0, nper, step=2)
        def _(j):
            u0 = base + j
            u1 = u0 + 1
            load(u1, idx1, cb1)
            wait_load(u0, idx0, cb0)
            scat(idx0, cb0)

            @pl.when(j + 2 < nper)
            def _():
                load(u0 + 2, idx0, cb0)

            wait_load(u1, idx1, cb1)
            scat(idx1, cb1)

        plsc.subcore_barrier()
        pltpu.sync_copy(accB.at[pl.ds(s * (_NB // 16), _NB // 16)],
                        ocg_hbm.at[c].at[pl.ds(s * (_NB // 16), _NB // 16)])

    return k(cpack, rows8, zb)


def _scatter(m_lo, m_hi, cpack, rows4, rows8, za, zb):
    magg_lo, magg_hi = _scatter_m(m_lo, m_hi, rows4, za)
    ocg = _scatter_c(cpack, rows8, zb)
    cagg = jnp.concatenate([ocg[0, 0:_NH], ocg[1, 0:_NH]], axis=0)
    return magg_lo[0:N], magg_hi[0:N], cagg


def kernel(time, mu_charge_t, mu_pos_t, gamma_coord, gamma_charge,
           edge_index, segment_ids, params):
    row = edge_index[0]
    col = edge_index[1]
    zpad = jnp.zeros((EPAD - E,), jnp.int32)
    rowpad = jnp.concatenate([row, zpad])
    colpad = jnp.concatenate([col, zpad])
    idx2 = jnp.concatenate([rowpad, colpad]).reshape(1, 2 * EPAD)
    rows4 = rowpad.reshape(_S1SUP, _S1CH // 128, 128)
    rows8 = rowpad.reshape(_S2SUP, _S2CH // 128, 128)
    za = jnp.zeros((_NA // 16, 32), jnp.float32)
    zb = jnp.zeros((_NB // 16, 16), jnp.float32)

    packed = _embed(mu_charge_t, time, mu_pos_t,
                    params['emb_in_w'], params['emb_in_b'])

    for p in params['layers']:
        w1cat = jnp.zeros((2 * PK, HID), jnp.float32)
        w1cat = w1cat.at[0:64].set(p['edge_w1'][0:64])
        w1cat = w1cat.at[PK:PK + 64].set(p['edge_w1'][64:128])
        aux_e = jnp.zeros((8, HID), jnp.float32)
        aux_e = aux_e.at[0].set(p['edge_w1'][128])
        aux_e = aux_e.at[1].set(p['edge_b1'])
        aux_e = aux_e.at[2].set(p['edge_b2'])
        aux_e = aux_e.at[3].set(p['coord_b1'])
        aux_e = aux_e.at[4].set(p['coord_w2'][:, 0])

        G = _gather(packed, idx2)
        m_lo, m_hi, cpack = _edge_mlp(G, w1cat, p['edge_w2'], p['coord_w1'],
                                      aux_e)
        magg_lo, magg_hi, cagg = _scatter(m_lo, m_hi, cpack, rows4, rows8, za, zb)

        aux_n = jnp.zeros((8, HID), jnp.float32)
        aux_n = aux_n.at[0].set(p['node_b1'])
        aux_n = aux_n.at[1].set(p['node_b2'])
        packed = _node_mlp(packed, magg_lo, magg_hi, cagg,
                           p['node_w1'], p['node_w2'], aux_n)

    ids3 = segment_ids.reshape(N // BN, 1, BN)
    idsc = segment_ids.reshape(N, 1)
    ss = _seg_sums(ids3, packed, mu_pos_t)
    aux_h = jnp.zeros((8, HID), jnp.float32)
    aux_h = aux_h.at[0].set(params['emb_out_w'][:, 0])
    aux_h = aux_h.at[1, 0].set(params['emb_out_b'][0])
    coord_pred, k_hat = _head(idsc, packed, mu_pos_t, mu_charge_t,
                              gamma_coord, gamma_charge, ss, aux_h)
    return coord_pred, k_hat


# speedup vs baseline: 1.6860x; 1.6860x over previous
import functools

import jax
import jax.numpy as jnp
import numpy as np
from jax import lax
from jax.experimental import pallas as pl
from jax.experimental.pallas import tpu as pltpu
from jax.experimental.pallas import tpu_sc as plsc

N = 50000
E = 800000
HID = 64
N_LAYERS = 4
NSEG = 500
SEGP = 512
PK = 128
EPAD = 802816
BE = 2048
BN = 1000

_CENTERS = np.linspace(-2.0, 2.0, 16, dtype=np.float32)
_GWIDTH = 4.0 / 16.0


def _silu(x):
    return x * jax.nn.sigmoid(x)


def _embed_body(mu_ref, t_ref, c_ref, w_ref, b_ref, out_ref):
    x = mu_ref[...]
    centers = (lax.broadcasted_iota(jnp.int32, (1, 16), 1).astype(jnp.float32)
               * (4.0 / 15.0) - 2.0)
    z = (x - centers) / _GWIDTH
    g = jnp.exp(-0.5 * z * z)
    g = g / jnp.clip(jnp.sum(jnp.abs(g), axis=1, keepdims=True), 1e-12, None)
    g = g * 2.0 - 1.0
    feat = jnp.concatenate([g, t_ref[...]], axis=1)
    h0 = jnp.dot(feat.astype(jnp.bfloat16), w_ref[...],
                 preferred_element_type=jnp.float32) + b_ref[...]
    out_ref[...] = jnp.concatenate(
        [h0, c_ref[...], jnp.zeros((BN, PK - 67), jnp.float32)], axis=1)


def _embed(mu_charge_t, time, mu_pos_t, w, b):
    grid = (N // BN,)
    return pl.pallas_call(
        _embed_body,
        grid=grid,
        in_specs=[
            pl.BlockSpec((BN, 1), lambda i: (i, 0)),
            pl.BlockSpec((BN, 1), lambda i: (i, 0)),
            pl.BlockSpec((BN, 3), lambda i: (i, 0)),
            pl.BlockSpec((17, HID), lambda i: (0, 0)),
            pl.BlockSpec((1, HID), lambda i: (0, 0)),
        ],
        out_specs=pl.BlockSpec((BN, PK), lambda i: (i, 0)),
        out_shape=jax.ShapeDtypeStruct((N, PK), jnp.float32),
    )(mu_charge_t, time, mu_pos_t, w, b.reshape(1, HID))


def _edge_body(g1_ref, g2_ref, w1_ref, w2_ref, wc1_ref, aux_ref,
               mlo_ref, mhi_ref, cp_ref):
    b = pl.program_id(0)
    g1 = g1_ref[...]
    g2 = g2_ref[...]
    diff = g1[:, 64:67] - g2[:, 64:67]
    radial = jnp.sum(diff * diff, axis=1, keepdims=True)
    cdiff = diff / (jnp.sqrt(radial) + 1.0)
    f = jnp.concatenate([g1, g2], axis=1)
    w1r = aux_ref[0:1, :]
    b1 = aux_ref[1:2, :]
    b2 = aux_ref[2:3, :]
    bc1 = aux_ref[3:4, :]
    wc2 = aux_ref[4:5, :]
    radial_b = radial.astype(jnp.bfloat16).astype(jnp.float32)
    a1 = _silu(jnp.dot(f.astype(jnp.bfloat16), w1_ref[...],
                       preferred_element_type=jnp.float32)
               + radial_b * w1r + b1)
    m = _silu(jnp.dot(a1.astype(jnp.bfloat16), w2_ref[...],
                      preferred_element_type=jnp.float32) + b2)
    t = _silu(jnp.dot(m.astype(jnp.bfloat16), wc1_ref[...],
                      preferred_element_type=jnp.float32) + bc1)
    phi = jnp.sum(t.astype(jnp.bfloat16).astype(jnp.float32) * wc2,
                  axis=1, keepdims=True)
    gidx = b * BE + lax.broadcasted_iota(jnp.int32, (BE, 1), 0)
    msk = (gidx < E).astype(jnp.float32)
    m = m * msk
    mlo_ref[...] = m[:, 0:32]
    mhi_ref[...] = m[:, 32:64]
    cp_ref[...] = jnp.concatenate(
        [cdiff * phi * msk, msk, jnp.zeros((BE, 12), jnp.float32)], axis=1)


def _edge_mlp(G, w1cat, w2, wc1, aux):
    nb = EPAD // BE
    return pl.pallas_call(
        _edge_body,
        grid=(nb,),
        in_specs=[
            pl.BlockSpec((BE, PK), lambda b: (b, 0)),
            pl.BlockSpec((BE, PK), lambda b, nb=nb: (b + nb, 0)),
            pl.BlockSpec((2 * PK, HID), lambda b: (0, 0)),
            pl.BlockSpec((HID, HID), lambda b: (0, 0)),
            pl.BlockSpec((HID, HID), lambda b: (0, 0)),
            pl.BlockSpec((8, HID), lambda b: (0, 0)),
        ],
        out_specs=[
            pl.BlockSpec((BE, 32), lambda b: (b, 0)),
            pl.BlockSpec((BE, 32), lambda b: (b, 0)),
            pl.BlockSpec((BE, 16), lambda b: (b, 0)),
        ],
        out_shape=[
            jax.ShapeDtypeStruct((EPAD, 32), jnp.float32),
            jax.ShapeDtypeStruct((EPAD, 32), jnp.float32),
            jax.ShapeDtypeStruct((EPAD, 16), jnp.float32),
        ],
    )(G, G, w1cat, w2, wc1, aux)


def _node_body(pk_ref, mlo_ref, mhi_ref, cg_ref, w1_ref, w2_ref, aux_ref, out_ref):
    pk = pk_ref[...]
    h = pk[:, 0:64]
    coord = pk[:, 64:67]
    cagg = cg_ref[...]
    cnt = jnp.clip(cagg[:, 3:4], 1.0, None)
    coord2 = coord + cagg[:, 0:3] / cnt
    f = jnp.concatenate([h, mlo_ref[...], mhi_ref[...]], axis=1)
    b1 = aux_ref[0:1, :]
    b2 = aux_ref[1:2, :]
    u = _silu(jnp.dot(f.astype(jnp.bfloat16), w1_ref[...],
                      preferred_element_type=jnp.float32) + b1)
    h2 = h + jnp.dot(u.astype(jnp.bfloat16), w2_ref[...],
                     preferred_element_type=jnp.float32) + b2
    out_ref[...] = jnp.concatenate(
        [h2, coord2, jnp.zeros((BN, PK - 67), jnp.float32)], axis=1)


def _node_mlp(packed, magg_lo, magg_hi, cagg, w1, w2, aux):
    return pl.pallas_call(
        _node_body,
        grid=(N // BN,),
        in_specs=[
            pl.BlockSpec((BN, PK), lambda i: (i, 0)),
            pl.BlockSpec((BN, 32), lambda i: (i, 0)),
            pl.BlockSpec((BN, 32), lambda i: (i, 0)),
            pl.BlockSpec((BN, 16), lambda i: (i, 0)),
            pl.BlockSpec((2 * HID, HID), lambda i: (0, 0)),
            pl.BlockSpec((HID, HID), lambda i: (0, 0)),
            pl.BlockSpec((8, HID), lambda i: (0, 0)),
        ],
        out_specs=pl.BlockSpec((BN, PK), lambda i: (i, 0)),
        out_shape=jax.ShapeDtypeStruct((N, PK), jnp.float32),
    )(packed, magg_lo, magg_hi, cagg, w1, w2, aux)


def _segA_body(ids_ref, pk_ref, mp_ref, acc_ref):
    b = pl.program_id(0)
    ids = ids_ref[0]
    eps0 = pk_ref[...][:, 64:67] - mp_ref[...]
    vals = jnp.concatenate(
        [eps0, jnp.ones((BN, 1), jnp.float32), jnp.zeros((BN, 4), jnp.float32)],
        axis=1)
    iota_col = lax.broadcasted_iota(jnp.int32, (SEGP, 1), 0)
    oh = (iota_col == ids).astype(jnp.float32)
    part = jnp.dot(oh, vals, preferred_element_type=jnp.float32)

    @pl.when(b == 0)
    def _():
        acc_ref[...] = part

    @pl.when(b > 0)
    def _():
        acc_ref[...] += part


def _seg_sums(ids3, packed, mu_pos_t):
    return pl.pallas_call(
        _segA_body,
        grid=(N // BN,),
        in_specs=[
            pl.BlockSpec((1, 1, BN), lambda i: (i, 0, 0)),
            pl.BlockSpec((BN, PK), lambda i: (i, 0)),
            pl.BlockSpec((BN, 3), lambda i: (i, 0)),
        ],
        out_specs=pl.BlockSpec((SEGP, 8), lambda i: (0, 0)),
        out_shape=jax.ShapeDtypeStruct((SEGP, 8), jnp.float32),
    )(ids3, packed, mu_pos_t)


def _segB_body(idsc_ref, pk_ref, mp_ref, mc_ref, gc_ref, gch_ref, ss_ref,
               aux_ref, cp_ref, kh_ref):
    ss = ss_ref[...]
    mean = ss[:, 0:3] / jnp.clip(ss[:, 3:4], 1.0, None)
    meanp = jnp.concatenate([mean, jnp.zeros((SEGP, 5), jnp.float32)], axis=1)
    iota_row = lax.broadcasted_iota(jnp.int32, (1, SEGP), 1)
    oh = (idsc_ref[...] == iota_row).astype(jnp.float32)
    mnode = jnp.dot(oh, meanp, preferred_element_type=jnp.float32)[:, 0:3]
    pk = pk_ref[...]
    mp = mp_ref[...]
    eps = jnp.clip(pk[:, 64:67] - mp - mnode, -10.0, 10.0)
    g = gc_ref[...]
    cp_ref[...] = mp / g - jnp.sqrt((1.0 - g) / g) * eps
    w0 = aux_ref[0:1, :]
    b0 = aux_ref[1:2, 0:1]
    hb = pk[:, 0:64].astype(jnp.bfloat16).astype(jnp.float32)
    mu_eps = jnp.clip(jnp.sum(hb * w0, axis=1, keepdims=True) + b0,
                      -10.0, 10.0)
    gch = gch_ref[...]
    kh_ref[...] = mc_ref[...] / gch - jnp.sqrt((1.0 - gch) / gch) * mu_eps


def _head(idsc, packed, mu_pos_t, mu_charge_t, gamma_coord, gamma_charge,
          seg_sums, aux):
    return pl.pallas_call(
        _segB_body,
        grid=(N // BN,),
        in_specs=[
            pl.BlockSpec((BN, 1), lambda i: (i, 0)),
            pl.BlockSpec((BN, PK), lambda i: (i, 0)),
            pl.BlockSpec((BN, 3), lambda i: (i, 0)),
            pl.BlockSpec((BN, 1), lambda i: (i, 0)),
            pl.BlockSpec((BN, 1), lambda i: (i, 0)),
            pl.BlockSpec((BN, 1), lambda i: (i, 0)),
            pl.BlockSpec((SEGP, 8), lambda i: (0, 0)),
            pl.BlockSpec((8, HID), lambda i: (0, 0)),
        ],
        out_specs=[
            pl.BlockSpec((BN, 3), lambda i: (i, 0)),
            pl.BlockSpec((BN, 1), lambda i: (i, 0)),
        ],
        out_shape=[
            jax.ShapeDtypeStruct((N, 3), jnp.float32),
            jax.ShapeDtypeStruct((N, 1), jnp.float32),
        ],
    )(idsc, packed, mu_pos_t, mu_charge_t, gamma_coord, gamma_charge,
      seg_sums, aux)


_GW = 128


def _gather(packed, idx2):
    n_idx = 2 * EPAD

    @functools.partial(
        pl.kernel,
        out_type=jax.ShapeDtypeStruct((n_idx, PK), jnp.float32),
        mesh=plsc.VectorSubcoreMesh(core_axis_name="c", subcore_axis_name="s"),
    )
    def k(x_hbm, i_hbm, o_hbm):
        def body(i_vmem, o_vmem):
            pltpu.sync_copy(x_hbm.at[i_vmem.at[0]], o_vmem)

        pltpu.emit_pipeline(
            body,
            grid=(n_idx // _GW,),
            in_specs=[pl.BlockSpec((1, _GW), index_map=lambda i: (0, i))],
            out_specs=[pl.BlockSpec((_GW, PK), index_map=lambda i: (i, 0))],
            core_axis_name=("c", "s"),
            dimension_semantics=(pltpu.PARALLEL,),
        )(i_hbm, o_hbm)

    return k(packed, idx2)


_NH = N // 2
_NA = 50048
_NB = 25088
_S1CH = 256
_S1SUP = EPAD // _S1CH
_S2CH = 512
_S2SUP = EPAD // _S2CH

_SC_PARAMS = pltpu.CompilerParams(use_tc_tiling_on_sc=False)


def _scatter_m(m_lo, m_hi, rows4, za):
    nper = _S1SUP // 16

    @functools.partial(
        pl.kernel,
        out_type=[
            jax.ShapeDtypeStruct((_NA, 32), jnp.float32),
            jax.ShapeDtypeStruct((_NA, 32), jnp.float32),
        ],
        mesh=plsc.VectorSubcoreMesh(core_axis_name="c", subcore_axis_name="s"),
        scratch_types=[
            pltpu.VMEM_SHARED((_NA, 32), jnp.float32),
            pltpu.VMEM((_S1CH // 128, 128), jnp.int32),
            pltpu.VMEM((_S1CH // 128, 128), jnp.int32),
            pltpu.VMEM((_S1CH, 32), jnp.float32),
            pltpu.VMEM((_S1CH, 32), jnp.float32),
            pltpu.SemaphoreType.DMA,
        ],
        compiler_params=_SC_PARAMS,
    )
    def k(mlo_hbm, mhi_hbm, rows_hbm, za_hbm, olo_hbm, ohi_hbm,
          accA, idx0, idx1, mb0, mb1, sem):
        c = lax.axis_index("c")
        s = lax.axis_index("s")
        pltpu.sync_copy(za_hbm, accA.at[pl.ds(s * (_NA // 16), _NA // 16)])
        plsc.subcore_barrier()

        def load(u, ib, mb):
            pltpu.async_copy(rows_hbm.at[u], ib, sem)

            @pl.when(c == 0)
            def _():
                pltpu.async_copy(mlo_hbm.at[pl.ds(u * _S1CH, _S1CH)], mb, sem)

            @pl.when(c == 1)
            def _():
                pltpu.async_copy(mhi_hbm.at[pl.ds(u * _S1CH, _S1CH)], mb, sem)

        def wait_load(u, ib, mb):
            pltpu.make_async_copy(rows_hbm.at[u], ib, sem).wait()
            pltpu.make_async_copy(mlo_hbm.at[pl.ds(u * _S1CH, _S1CH)], mb,
                                  sem).wait()

        def scat(ib, mb):
            @pl.loop(0, _S1CH // 128)
            def _(kk):
                pltpu.sync_copy(mb.at[pl.ds(kk * 128, 128)],
                                accA.at[ib.at[kk]], add=True)

        base = s * nper
        load(base, idx0, mb0)

        @pl.loop(0, nper, step=2)
        def _(j):
            u0 = base + j
            u1 = u0 + 1
            load(u1, idx1, mb1)
            wait_load(u0, idx0, mb0)
            scat(idx0, mb0)

            @pl.when(j + 2 < nper)
            def _():
                load(u0 + 2, idx0, mb0)

            wait_load(u1, idx1, mb1)
            scat(idx1, mb1)

        plsc.subcore_barrier()

        @pl.when(c == 0)
        def _():
            pltpu.sync_copy(accA.at[pl.ds(s * (_NA // 16), _NA // 16)],
                            olo_hbm.at[pl.ds(s * (_NA // 16), _NA // 16)])

        @pl.when(c == 1)
        def _():
            pltpu.sync_copy(accA.at[pl.ds(s * (_NA // 16), _NA // 16)],
                            ohi_hbm.at[pl.ds(s * (_NA // 16), _NA // 16)])

    return k(m_lo, m_hi, rows4, za)


def _scatter_c(cpack, rows8, zb):
    nper = _S2SUP // 16

    @functools.partial(
        pl.kernel,
        out_type=jax.ShapeDtypeStruct((2, _NB, 16), jnp.float32),
        mesh=plsc.VectorSubcoreMesh(core_axis_name="c", subcore_axis_name="s"),
        scratch_types=[
            pltpu.VMEM_SHARED((_NB, 16), jnp.float32),
            pltpu.VMEM((_S2CH // 128, 128), jnp.int32),
            pltpu.VMEM((_S2CH // 128, 128), jnp.int32),
            pltpu.VMEM((_S2CH // 128, 128), jnp.int32),
            pltpu.VMEM((_S2CH, 16), jnp.float32),
            pltpu.VMEM((_S2CH, 16), jnp.float32),
            pltpu.SemaphoreType.DMA,
        ],
        compiler_params=_SC_PARAMS,
    )
    def k(cp_hbm, rows_hbm, zb_hbm, ocg_hbm,
          accB, idx0, idx1, idxt, cb0, cb1, sem):
        c = lax.axis_index("c")
        s = lax.axis_index("s")
        pltpu.sync_copy(zb_hbm, accB.at[pl.ds(s * (_NB // 16), _NB // 16)])
        plsc.subcore_barrier()
        nbase = c * _NH

        def load(u, ib, cb):
            pltpu.async_copy(rows_hbm.at[u], ib, sem)
            pltpu.async_copy(cp_hbm.at[pl.ds(u * _S2CH, _S2CH)], cb, sem)

        def wait_load(u, ib, cb):
            pltpu.make_async_copy(rows_hbm.at[u], ib, sem).wait()
            pltpu.make_async_copy(cp_hbm.at[pl.ds(u * _S2CH, _S2CH)], cb,
                                  sem).wait()

        def scat(ib, cb):
            @pl.loop(0, _S2CH // 128)
            def _(kk):
                @pl.loop(0, 8)
                def _(t):
                    v = ib[kk, pl.ds(t * 16, 16)]
                    l = v - nbase
                    ok = (l >= 0) & (l < _NH)
                    idxt[kk, pl.ds(t * 16, 16)] = jnp.where(ok, l, _NB - 1)

            @pl.loop(0, _S2CH // 128)
            def _(kk):
                pltpu.sync_copy(cb.at[pl.ds(kk * 128, 128)],
                                accB.at[idxt.at[kk]], add=True)

        base = s * nper
        load(base, idx0, cb0)

        @pl.loop(0, nper, step=2)
        def _(j):
            u0 = base + j
            u1 = u0 + 1
            load(u1, idx1, cb1)
            wait_load(u0, idx0, cb0)
            scat(idx0, cb0)

            @pl.when(j + 2 < nper)
            def _():
                load(u0 + 2, idx0, cb0)

            wait_load(u1, idx1, cb1)
            scat(idx1, cb1)

        plsc.subcore_barrier()
        pltpu.sync_copy(accB.at[pl.ds(s * (_NB // 16), _NB // 16)],
                        ocg_hbm.at[c].at[pl.ds(s * (_NB // 16), _NB // 16)])

    return k(cpack, rows8, zb)


def _scatter(m_lo, m_hi, cpack, rows4, rows8, za, zb):
    magg_lo, magg_hi = _scatter_m(m_lo, m_hi, rows4, za)
    ocg = _scatter_c(cpack, rows8, zb)
    cagg = jnp.concatenate([ocg[0, 0:_NH], ocg[1, 0:_NH]], axis=0)
    return magg_lo[0:N], magg_hi[0:N], cagg


def kernel(time, mu_charge_t, mu_pos_t, gamma_coord, gamma_charge,
           edge_index, segment_ids, params):
    row = edge_index[0]
    col = edge_index[1]
    zpad = jnp.zeros((EPAD - E,), jnp.int32)
    rowpad = jnp.concatenate([row, zpad])
    colpad = jnp.concatenate([col, zpad])
    idx2 = jnp.concatenate([rowpad, colpad]).reshape(1, 2 * EPAD)
    rows4 = rowpad.reshape(_S1SUP, _S1CH // 128, 128)
    rows8 = rowpad.reshape(_S2SUP, _S2CH // 128, 128)
    za = jnp.zeros((_NA // 16, 32), jnp.float32)
    zb = jnp.zeros((_NB // 16, 16), jnp.float32)

    packed = _embed(mu_charge_t, time, mu_pos_t,
                    params['emb_in_w'].astype(jnp.bfloat16),
                    params['emb_in_b'])

    for p in params['layers']:
        w1cat = jnp.zeros((2 * PK, HID), jnp.float32)
        w1cat = w1cat.at[0:64].set(p['edge_w1'][0:64])
        w1cat = w1cat.at[PK:PK + 64].set(p['edge_w1'][64:128])
        aux_e = jnp.zeros((8, HID), jnp.float32)
        aux_e = aux_e.at[0].set(
            p['edge_w1'][128].astype(jnp.bfloat16).astype(jnp.float32))
        aux_e = aux_e.at[1].set(p['edge_b1'])
        aux_e = aux_e.at[2].set(p['edge_b2'])
        aux_e = aux_e.at[3].set(p['coord_b1'])
        aux_e = aux_e.at[4].set(
            p['coord_w2'][:, 0].astype(jnp.bfloat16).astype(jnp.float32))

        G = _gather(packed, idx2)
        m_lo, m_hi, cpack = _edge_mlp(G, w1cat.astype(jnp.bfloat16),
                                      p['edge_w2'].astype(jnp.bfloat16),
                                      p['coord_w1'].astype(jnp.bfloat16),
                                      aux_e)
        magg_lo, magg_hi, cagg = _scatter(m_lo, m_hi, cpack, rows4, rows8, za, zb)

        aux_n = jnp.zeros((8, HID), jnp.float32)
        aux_n = aux_n.at[0].set(p['node_b1'])
        aux_n = aux_n.at[1].set(p['node_b2'])
        packed = _node_mlp(packed, magg_lo, magg_hi, cagg,
                           p['node_w1'].astype(jnp.bfloat16),
                           p['node_w2'].astype(jnp.bfloat16), aux_n)

    ids3 = segment_ids.reshape(N // BN, 1, BN)
    idsc = segment_ids.reshape(N, 1)
    ss = _seg_sums(ids3, packed, mu_pos_t)
    aux_h = jnp.zeros((8, HID), jnp.float32)
    aux_h = aux_h.at[0].set(
        params['emb_out_w'][:, 0].astype(jnp.bfloat16).astype(jnp.float32))
    aux_h = aux_h.at[1, 0].set(params['emb_out_b'][0])
    coord_pred, k_hat = _head(idsc, packed, mu_pos_t, mu_charge_t,
                              gamma_coord, gamma_charge, ss, aux_h)
    return coord_pred, k_hat
